# baseline (speedup 1.0000x reference)
import functools

import jax
import jax.numpy as jnp
from jax import lax
from jax.experimental import pallas as pl
from jax.experimental.pallas import tpu as pltpu
from jax.experimental.pallas import tpu_sc as plsc

N = 10000
E = 320000
F_IN = 128
HEADS = 8
HIDS = 8
NCLS = 40

NC = 2
NS = 16
NW = NC * NS
CK = 80
EW = E // NW
NCHUNK = EW // CK
ROWS_PER_TILE = N // NS

BN = 1000

NB = 4

_mesh = plsc.VectorSubcoreMesh(core_axis_name="c", subcore_axis_name="s")


def _pipeline(fire_gather, wait_gather, fire_scatter, wait_scatter, compute):
    assert NCHUNK % NB == 1
    for b in range(NB - 1):
        fire_gather(b, b)
    for j0 in range(NB):
        fire_gather(j0 + NB - 1, (j0 + NB - 1) % NB)
        wait_gather(j0, j0)
        compute(j0, j0)
        fire_scatter(j0, j0)

    def qbody(q, carry):
        j0 = NB * q
        for b in range(NB):
            j = j0 + b
            jn = jnp.minimum(j + NB - 1, NCHUNK - 1)
            fire_gather(jn, (b + NB - 1) % NB)
            wait_gather(j, b)
            wait_scatter(j, b)
            compute(j, b)
            fire_scatter(j, b)
        return carry

    lax.fori_loop(1, (NCHUNK - 1) // NB, qbody, 0)
    jt = NCHUNK - 1
    bt = jt % NB
    wait_gather(jt, bt)
    wait_scatter(jt, bt)
    compute(jt, bt)
    fire_scatter(jt, bt)
    for bb in range(1, NB - 1):
        wait_gather(jt, bb)
    for bb in range(NB):
        wait_scatter(jt, bb)
_sc_params = pltpu.CompilerParams(use_tc_tiling_on_sc=False,
                                  needs_layout_passes=False)


def _tc_prep1(x_ref, w1_ref, asrc_ref, adst_ref, st_ref, dt_ref):
    h = jnp.dot(x_ref[...], w1_ref[...], preferred_element_type=jnp.float32)
    a_s = jnp.dot(h, asrc_ref[...], preferred_element_type=jnp.float32)
    a_d = jnp.dot(h, adst_ref[...], preferred_element_type=jnp.float32)
    z = jnp.zeros((BN, 8), jnp.float32)
    st_ref[...] = jnp.concatenate([a_s, z, h], axis=1)
    dt_ref[...] = jnp.concatenate([a_d, z], axis=1)


def _tc_mid(a0_ref, a1_ref, b1_ref, w2_ref, as2_ref, ad2_ref, r8_ref,
            st_ref, s2_ref, d2_ref):
    a0 = a0_ref[...]
    a1 = a1_ref[...]
    den = a0[:, 0:8] + a1[:, 0:8]
    msum = a0[:, 16:80] + a1[:, 16:80]
    recip = 1.0 / (den + 1e-16)
    expand = jnp.dot(recip, r8_ref[...], preferred_element_type=jnp.float32)
    x1 = msum * expand + b1_ref[...]
    x1 = jnp.where(x1 > 0, x1, jnp.exp(x1) - 1.0)
    h2 = jnp.dot(x1, w2_ref[...], preferred_element_type=jnp.float32)
    s2_ref[...] = jnp.sum(h2 * as2_ref[...], axis=1, keepdims=True)
    d2_ref[...] = jnp.sum(h2 * ad2_ref[...], axis=1, keepdims=True)
    st_ref[...] = jnp.concatenate(
        [jnp.ones((BN, 1), jnp.float32), jnp.zeros((BN, 7), jnp.float32), h2],
        axis=1)


def _tc_final(a0_ref, a1_ref, b2_ref, out_ref):
    a0 = a0_ref[...]
    a1 = a1_ref[...]
    den = a0[:, 0:1] + a1[:, 0:1]
    msg = a0[:, 8:48] + a1[:, 8:48]
    out_ref[...] = msg / (den + 1e-16) + b2_ref[...]


def _sc_layer1(st_ref, adt_ref, adjs_ref, adjd_ref, zero_ref, out_ref,
               acc, sbufs, cbufs, dbufs, idxs_v, idxd_v, gss, gds, sss):
    cid = lax.axis_index("c")
    sid = lax.axis_index("s")
    w = sid * NC + cid

    r0 = sid * ROWS_PER_TILE
    pltpu.sync_copy(zero_ref.at[pl.ds(r0, ROWS_PER_TILE)],
                    acc.at[pl.ds(r0, ROWS_PER_TILE)])
    pltpu.sync_copy(adjs_ref.at[pl.ds(w * NCHUNK, NCHUNK)], idxs_v)
    pltpu.sync_copy(adjd_ref.at[pl.ds(w * NCHUNK, NCHUNK)], idxd_v)
    plsc.subcore_barrier()

    iota = lax.iota(jnp.int32, 16)

    def fire_gather(j, b):
        pltpu.async_copy(st_ref.at[idxs_v.at[j]], sbufs[b], gss[b])
        pltpu.async_copy(adt_ref.at[idxd_v.at[j]], dbufs[b], gds[b])

    def wait_gather(j, b):
        pltpu.make_async_copy(st_ref.at[idxs_v.at[j]], sbufs[b], gss[b]).wait()
        pltpu.make_async_copy(adt_ref.at[idxd_v.at[j]], dbufs[b], gds[b]).wait()

    def fire_scatter(j, b):
        pltpu.async_copy(cbufs[b], acc.at[idxd_v.at[j]], sss[b], add=True)

    def wait_scatter(j, b):
        pltpu.make_async_copy(cbufs[b], acc.at[idxd_v.at[j]], sss[b]).wait()

    idx01 = iota >> 3

    def compute(j, b):
        sbuf, dbuf, cbuf = sbufs[b], dbufs[b], cbufs[b]

        def edge_body(e, carry):
            s0 = sbuf[e, pl.ds(0, 16)]
            d0 = dbuf[e, pl.ds(0, 16)]
            zz = s0 + d0
            ee = jnp.exp(jnp.maximum(zz, 0.2 * zz))
            cbuf[e, pl.ds(0, 16)] = ee
            for k in range(4):
                mk = ee.at[idx01 + 2 * k].get(mode="promise_in_bounds")
                hk = sbuf[e, pl.ds(16 + 16 * k, 16)]
                cbuf[e, pl.ds(16 + 16 * k, 16)] = hk * mk
            return carry

        lax.fori_loop(0, CK, edge_body, 0, unroll=4)

    _pipeline(fire_gather, wait_gather, fire_scatter, wait_scatter, compute)
    plsc.subcore_barrier()
    pltpu.sync_copy(acc.at[pl.ds(r0, ROWS_PER_TILE)],
                    out_ref.at[cid, pl.ds(r0, ROWS_PER_TILE)])


def _sc_layer2(st_ref, as2_ref, ad2_ref, adjs_ref, adjd_ref, zero_ref,
               out_ref, acc, sbufs, cbufs, asrc_v, adst_v,
               idxs_v, idxd_v, gss, sss):
    cid = lax.axis_index("c")
    sid = lax.axis_index("s")
    w = sid * NC + cid

    r0 = sid * ROWS_PER_TILE
    pltpu.sync_copy(zero_ref.at[pl.ds(r0, ROWS_PER_TILE)],
                    acc.at[pl.ds(r0, ROWS_PER_TILE)])
    pltpu.sync_copy(as2_ref, asrc_v)
    pltpu.sync_copy(ad2_ref, adst_v)
    pltpu.sync_copy(adjs_ref.at[pl.ds(w * NCHUNK, NCHUNK)], idxs_v)
    pltpu.sync_copy(adjd_ref.at[pl.ds(w * NCHUNK, NCHUNK)], idxd_v)
    plsc.subcore_barrier()

    def fire_gather(j, b):
        pltpu.async_copy(st_ref.at[idxs_v.at[j]], sbufs[b], gss[b])

    def wait_gather(j, b):
        pltpu.make_async_copy(st_ref.at[idxs_v.at[j]], sbufs[b], gss[b]).wait()

    def fire_scatter(j, b):
        pltpu.async_copy(cbufs[b], acc.at[idxd_v.at[j]], sss[b], add=True)

    def wait_scatter(j, b):
        pltpu.make_async_copy(cbufs[b], acc.at[idxd_v.at[j]], sss[b]).wait()

    def compute(j, b):
        sbuf, cbuf = sbufs[b], cbufs[b]
        for g in range(5):
            is_g = idxs_v[j, pl.ds(g * 16, 16)]
            id_g = idxd_v[j, pl.ds(g * 16, 16)]
            a_s = plsc.load_gather(asrc_v, [is_g])
            a_d = plsc.load_gather(adst_v, [id_g])
            zz = a_s + a_d
            ee16 = jnp.exp(jnp.maximum(zz, 0.2 * zz))

            def edge_body(i, carry):
                e = g * 16 + i
                ii = jnp.full((16,), i, jnp.int32)
                mult = ee16.at[ii].get(mode="promise_in_bounds")
                for k in range(3):
                    rk = sbuf[e, pl.ds(16 * k, 16)]
                    cbuf[e, pl.ds(16 * k, 16)] = rk * mult
                return carry

            lax.fori_loop(0, 16, edge_body, 0, unroll=4)

    _pipeline(fire_gather, wait_gather, fire_scatter, wait_scatter, compute)
    plsc.subcore_barrier()
    pltpu.sync_copy(acc.at[pl.ds(r0, ROWS_PER_TILE)],
                    out_ref.at[cid, pl.ds(r0, ROWS_PER_TILE)])


def kernel(data, adj, W1, a_src1, a_dst1, b1, W2, a_src2, a_dst2, b2):
    f32 = jnp.float32
    eye8 = jnp.eye(8, dtype=f32)
    A_src = (eye8[:, None, :] * a_src1[:, :, None]).reshape(64, 8)
    A_dst = (eye8[:, None, :] * a_dst1[:, :, None]).reshape(64, 8)
    R8 = jnp.repeat(eye8, 8, axis=1)

    grid = (N // BN,)
    bs = lambda shp: pl.BlockSpec(shp, lambda i: (i, 0))
    bfull = lambda shp: pl.BlockSpec(shp, lambda i: (0, 0))

    st1, adst1 = pl.pallas_call(
        _tc_prep1,
        grid=grid,
        in_specs=[bs((BN, F_IN)), bfull((F_IN, 64)), bfull((64, 8)),
                  bfull((64, 8))],
        out_specs=[bs((BN, 80)), bs((BN, 16))],
        out_shape=[jax.ShapeDtypeStruct((N, 80), f32),
                   jax.ShapeDtypeStruct((N, 16), f32)],
    )(data, W1, A_src, A_dst)

    adj_s = adj[0].reshape(E // CK, CK)
    adj_d = adj[1].reshape(E // CK, CK)
    zeros80 = jnp.zeros((N, 80), f32)
    zeros48 = jnp.zeros((N, 48), f32)

    sc1 = pl.kernel(
        _sc_layer1,
        out_type=jax.ShapeDtypeStruct((2, N, 80), f32),
        mesh=_mesh,
        compiler_params=_sc_params,
        scratch_types=[
            pltpu.VMEM_SHARED((N, 80), f32),
            [pltpu.VMEM((CK, 80), f32)] * NB,
            [pltpu.VMEM((CK, 80), f32)] * NB,
            [pltpu.VMEM((CK, 16), f32)] * NB,
            pltpu.VMEM((NCHUNK, CK), jnp.int32),
            pltpu.VMEM((NCHUNK, CK), jnp.int32),
            [pltpu.SemaphoreType.DMA] * NB,
            [pltpu.SemaphoreType.DMA] * NB,
            [pltpu.SemaphoreType.DMA] * NB,
        ],
    )
    acc1 = sc1(st1, adst1, adj_s, adj_d, zeros80)

    st2, asrc2, adst2 = pl.pallas_call(
        _tc_mid,
        grid=grid,
        in_specs=[bs((BN, 80)), bs((BN, 80)), bfull((1, 64)), bfull((64, 40)),
                  bfull((1, 40)), bfull((1, 40)), bfull((8, 64))],
        out_specs=[bs((BN, 48)), bs((BN, 1)), bs((BN, 1))],
        out_shape=[jax.ShapeDtypeStruct((N, 48), f32),
                   jax.ShapeDtypeStruct((N, 1), f32),
                   jax.ShapeDtypeStruct((N, 1), f32)],
    )(acc1[0], acc1[1], b1.reshape(1, 64), W2, a_src2, a_dst2, R8)

    sc2 = pl.kernel(
        _sc_layer2,
        out_type=jax.ShapeDtypeStruct((2, N, 48), f32),
        mesh=_mesh,
        compiler_params=_sc_params,
        scratch_types=[
            pltpu.VMEM_SHARED((N, 48), f32),
            [pltpu.VMEM((CK, 48), f32)] * NB,
            [pltpu.VMEM((CK, 48), f32)] * NB,
            pltpu.VMEM((N,), f32),
            pltpu.VMEM((N,), f32),
            pltpu.VMEM((NCHUNK, CK), jnp.int32),
            pltpu.VMEM((NCHUNK, CK), jnp.int32),
            [pltpu.SemaphoreType.DMA] * NB,
            [pltpu.SemaphoreType.DMA] * NB,
        ],
    )
    acc2 = sc2(st2, asrc2.reshape(N), adst2.reshape(N), adj_s, adj_d, zeros48)

    out = pl.pallas_call(
        _tc_final,
        grid=grid,
        in_specs=[bs((BN, 48)), bs((BN, 48)), bfull((1, 40))],
        out_specs=bs((BN, 40)),
        out_shape=jax.ShapeDtypeStruct((N, 40), f32),
    )(acc2[0], acc2[1], b2.reshape(1, 40))
    return out


# speedup vs baseline: 2.4463x; 2.4463x over previous
import functools

import jax
import jax.numpy as jnp
from jax import lax
from jax.experimental import pallas as pl
from jax.experimental.pallas import tpu as pltpu
from jax.experimental.pallas import tpu_sc as plsc

N = 10000
E = 320000
F_IN = 128
HEADS = 8
HIDS = 8
NCLS = 40

NC = 2
NS = 16
NW = NC * NS
CK = 80
EW = E // NW
NCHUNK = EW // CK
ROWS_PER_TILE = N // NS

BN = 1000

NB = 4

_mesh = plsc.VectorSubcoreMesh(core_axis_name="c", subcore_axis_name="s")


def _pipeline(fire_gather, wait_gather, fire_scatter, wait_scatter, compute):
    assert NCHUNK % NB == 1
    for b in range(NB - 1):
        fire_gather(b, b)
    for j0 in range(NB):
        fire_gather(j0 + NB - 1, (j0 + NB - 1) % NB)
        wait_gather(j0, j0)
        compute(j0, j0)
        fire_scatter(j0, j0)

    def qbody(q, carry):
        j0 = NB * q
        for b in range(NB):
            j = j0 + b
            jn = jnp.minimum(j + NB - 1, NCHUNK - 1)
            fire_gather(jn, (b + NB - 1) % NB)
            wait_gather(j, b)
            wait_scatter(j, b)
            compute(j, b)
            fire_scatter(j, b)
        return carry

    lax.fori_loop(1, (NCHUNK - 1) // NB, qbody, 0)
    jt = NCHUNK - 1
    bt = jt % NB
    wait_gather(jt, bt)
    wait_scatter(jt, bt)
    compute(jt, bt)
    fire_scatter(jt, bt)
    for bb in range(1, NB - 1):
        wait_gather(jt, bb)
    for bb in range(NB):
        wait_scatter(jt, bb)
_sc_params = pltpu.CompilerParams(use_tc_tiling_on_sc=False,
                                  needs_layout_passes=False)


def _tc_prep1(x_ref, w1_ref, asrc_ref, adst_ref, st_ref, dt_ref):
    h = jnp.dot(x_ref[...], w1_ref[...], preferred_element_type=jnp.float32)
    a_s = jnp.dot(h, asrc_ref[...], preferred_element_type=jnp.float32)
    a_d = jnp.dot(h, adst_ref[...], preferred_element_type=jnp.float32)
    z = jnp.zeros((BN, 8), jnp.float32)
    st_ref[...] = jnp.concatenate([a_s, z, h], axis=1)
    dt_ref[...] = jnp.concatenate([a_d, z], axis=1)


def _tc_mid(a0_ref, a1_ref, b1_ref, w2_ref, as2_ref, ad2_ref, r8_ref,
            st_ref, s2_ref, d2_ref):
    a0 = a0_ref[...]
    a1 = a1_ref[...]
    den = a0[:, 0:8] + a1[:, 0:8]
    msum = a0[:, 16:80] + a1[:, 16:80]
    recip = 1.0 / (den + 1e-16)
    expand = jnp.dot(recip, r8_ref[...], preferred_element_type=jnp.float32)
    x1 = msum * expand + b1_ref[...]
    x1 = jnp.where(x1 > 0, x1, jnp.exp(x1) - 1.0)
    h2 = jnp.dot(x1, w2_ref[...], preferred_element_type=jnp.float32)
    s2_ref[...] = jnp.sum(h2 * as2_ref[...], axis=1, keepdims=True)
    d2_ref[...] = jnp.sum(h2 * ad2_ref[...], axis=1, keepdims=True)
    st_ref[...] = jnp.concatenate(
        [jnp.ones((BN, 1), jnp.float32), jnp.zeros((BN, 7), jnp.float32), h2],
        axis=1)


def _tc_final(a0_ref, a1_ref, b2_ref, out_ref):
    a0 = a0_ref[...]
    a1 = a1_ref[...]
    den = a0[:, 0:1] + a1[:, 0:1]
    msg = a0[:, 8:48] + a1[:, 8:48]
    out_ref[...] = msg / (den + 1e-16) + b2_ref[...]


def _sc_layer1(st_ref, adt_ref, adjs_ref, adjd_ref, zero_ref, out_ref,
               acc, sbufs, cbufs, dbufs, idxs_v, idxd_v, gss, gds, sss):
    cid = lax.axis_index("c")
    sid = lax.axis_index("s")
    w = sid * NC + cid

    r0 = sid * ROWS_PER_TILE
    pltpu.sync_copy(zero_ref.at[pl.ds(r0, ROWS_PER_TILE)],
                    acc.at[pl.ds(r0, ROWS_PER_TILE)])
    pltpu.sync_copy(adjs_ref.at[pl.ds(w * NCHUNK, NCHUNK)], idxs_v)
    pltpu.sync_copy(adjd_ref.at[pl.ds(w * NCHUNK, NCHUNK)], idxd_v)
    plsc.subcore_barrier()

    iota = lax.iota(jnp.int32, 16)

    def fire_gather(j, b):
        pltpu.async_copy(st_ref.at[idxs_v.at[j]], sbufs[b], gss[b])
        pltpu.async_copy(adt_ref.at[idxd_v.at[j]], dbufs[b], gds[b])

    def wait_gather(j, b):
        pltpu.make_async_copy(st_ref.at[idxs_v.at[j]], sbufs[b], gss[b]).wait()
        pltpu.make_async_copy(adt_ref.at[idxd_v.at[j]], dbufs[b], gds[b]).wait()

    def fire_scatter(j, b):
        pltpu.async_copy(cbufs[b], acc.at[idxd_v.at[j]], sss[b], add=True)

    def wait_scatter(j, b):
        pltpu.make_async_copy(cbufs[b], acc.at[idxd_v.at[j]], sss[b]).wait()

    idx01 = iota >> 3

    def compute(j, b):
        sbuf, dbuf, cbuf = sbufs[b], dbufs[b], cbufs[b]

        @plsc.parallel_loop(0, CK, unroll=4)
        def edge_body(e):
            s0 = sbuf[e, pl.ds(0, 16)]
            d0 = dbuf[e, pl.ds(0, 16)]
            zz = s0 + d0
            ee = jnp.exp(jnp.maximum(zz, 0.2 * zz))
            cbuf[e, pl.ds(0, 16)] = ee
            for k in range(4):
                mk = ee.at[idx01 + 2 * k].get(mode="promise_in_bounds")
                hk = sbuf[e, pl.ds(16 + 16 * k, 16)]
                cbuf[e, pl.ds(16 + 16 * k, 16)] = hk * mk

    _pipeline(fire_gather, wait_gather, fire_scatter, wait_scatter, compute)
    plsc.subcore_barrier()
    pltpu.sync_copy(acc.at[pl.ds(r0, ROWS_PER_TILE)],
                    out_ref.at[cid, pl.ds(r0, ROWS_PER_TILE)])


def _sc_layer2(st_ref, as2_ref, ad2_ref, adjs_ref, adjd_ref, zero_ref,
               out_ref, acc, sbufs, cbufs, asrc_v, adst_v,
               idxs_v, idxd_v, gss, sss):
    cid = lax.axis_index("c")
    sid = lax.axis_index("s")
    w = sid * NC + cid

    r0 = sid * ROWS_PER_TILE
    pltpu.sync_copy(zero_ref.at[pl.ds(r0, ROWS_PER_TILE)],
                    acc.at[pl.ds(r0, ROWS_PER_TILE)])
    pltpu.sync_copy(as2_ref, asrc_v)
    pltpu.sync_copy(ad2_ref, adst_v)
    pltpu.sync_copy(adjs_ref.at[pl.ds(w * NCHUNK, NCHUNK)], idxs_v)
    pltpu.sync_copy(adjd_ref.at[pl.ds(w * NCHUNK, NCHUNK)], idxd_v)
    plsc.subcore_barrier()

    def fire_gather(j, b):
        pltpu.async_copy(st_ref.at[idxs_v.at[j]], sbufs[b], gss[b])

    def wait_gather(j, b):
        pltpu.make_async_copy(st_ref.at[idxs_v.at[j]], sbufs[b], gss[b]).wait()

    def fire_scatter(j, b):
        pltpu.async_copy(cbufs[b], acc.at[idxd_v.at[j]], sss[b], add=True)

    def wait_scatter(j, b):
        pltpu.make_async_copy(cbufs[b], acc.at[idxd_v.at[j]], sss[b]).wait()

    def compute(j, b):
        sbuf, cbuf = sbufs[b], cbufs[b]
        for g in range(5):
            is_g = idxs_v[j, pl.ds(g * 16, 16)]
            id_g = idxd_v[j, pl.ds(g * 16, 16)]
            a_s = plsc.load_gather(asrc_v, [is_g])
            a_d = plsc.load_gather(adst_v, [id_g])
            zz = a_s + a_d
            ee16 = jnp.exp(jnp.maximum(zz, 0.2 * zz))

            @plsc.parallel_loop(0, 16, unroll=4)
            def edge_body(i):
                e = g * 16 + i
                ii = jnp.full((16,), i, jnp.int32)
                mult = ee16.at[ii].get(mode="promise_in_bounds")
                for k in range(3):
                    rk = sbuf[e, pl.ds(16 * k, 16)]
                    cbuf[e, pl.ds(16 * k, 16)] = rk * mult

    _pipeline(fire_gather, wait_gather, fire_scatter, wait_scatter, compute)
    plsc.subcore_barrier()
    pltpu.sync_copy(acc.at[pl.ds(r0, ROWS_PER_TILE)],
                    out_ref.at[cid, pl.ds(r0, ROWS_PER_TILE)])


def kernel(data, adj, W1, a_src1, a_dst1, b1, W2, a_src2, a_dst2, b2):
    f32 = jnp.float32
    eye8 = jnp.eye(8, dtype=f32)
    A_src = (eye8[:, None, :] * a_src1[:, :, None]).reshape(64, 8)
    A_dst = (eye8[:, None, :] * a_dst1[:, :, None]).reshape(64, 8)
    R8 = jnp.repeat(eye8, 8, axis=1)

    grid = (N // BN,)
    bs = lambda shp: pl.BlockSpec(shp, lambda i: (i, 0))
    bfull = lambda shp: pl.BlockSpec(shp, lambda i: (0, 0))

    st1, adst1 = pl.pallas_call(
        _tc_prep1,
        grid=grid,
        in_specs=[bs((BN, F_IN)), bfull((F_IN, 64)), bfull((64, 8)),
                  bfull((64, 8))],
        out_specs=[bs((BN, 80)), bs((BN, 16))],
        out_shape=[jax.ShapeDtypeStruct((N, 80), f32),
                   jax.ShapeDtypeStruct((N, 16), f32)],
    )(data, W1, A_src, A_dst)

    adj_s = adj[0].reshape(E // CK, CK)
    adj_d = adj[1].reshape(E // CK, CK)
    zeros80 = jnp.zeros((N, 80), f32)
    zeros48 = jnp.zeros((N, 48), f32)

    sc1 = pl.kernel(
        _sc_layer1,
        out_type=jax.ShapeDtypeStruct((2, N, 80), f32),
        mesh=_mesh,
        compiler_params=_sc_params,
        scratch_types=[
            pltpu.VMEM_SHARED((N, 80), f32),
            [pltpu.VMEM((CK, 80), f32)] * NB,
            [pltpu.VMEM((CK, 80), f32)] * NB,
            [pltpu.VMEM((CK, 16), f32)] * NB,
            pltpu.VMEM((NCHUNK, CK), jnp.int32),
            pltpu.VMEM((NCHUNK, CK), jnp.int32),
            [pltpu.SemaphoreType.DMA] * NB,
            [pltpu.SemaphoreType.DMA] * NB,
            [pltpu.SemaphoreType.DMA] * NB,
        ],
    )
    acc1 = sc1(st1, adst1, adj_s, adj_d, zeros80)

    st2, asrc2, adst2 = pl.pallas_call(
        _tc_mid,
        grid=grid,
        in_specs=[bs((BN, 80)), bs((BN, 80)), bfull((1, 64)), bfull((64, 40)),
                  bfull((1, 40)), bfull((1, 40)), bfull((8, 64))],
        out_specs=[bs((BN, 48)), bs((BN, 1)), bs((BN, 1))],
        out_shape=[jax.ShapeDtypeStruct((N, 48), f32),
                   jax.ShapeDtypeStruct((N, 1), f32),
                   jax.ShapeDtypeStruct((N, 1), f32)],
    )(acc1[0], acc1[1], b1.reshape(1, 64), W2, a_src2, a_dst2, R8)

    sc2 = pl.kernel(
        _sc_layer2,
        out_type=jax.ShapeDtypeStruct((2, N, 48), f32),
        mesh=_mesh,
        compiler_params=_sc_params,
        scratch_types=[
            pltpu.VMEM_SHARED((N, 48), f32),
            [pltpu.VMEM((CK, 48), f32)] * NB,
            [pltpu.VMEM((CK, 48), f32)] * NB,
            pltpu.VMEM((N,), f32),
            pltpu.VMEM((N,), f32),
            pltpu.VMEM((NCHUNK, CK), jnp.int32),
            pltpu.VMEM((NCHUNK, CK), jnp.int32),
            [pltpu.SemaphoreType.DMA] * NB,
            [pltpu.SemaphoreType.DMA] * NB,
        ],
    )
    acc2 = sc2(st2, asrc2.reshape(N), adst2.reshape(N), adj_s, adj_d, zeros48)

    out = pl.pallas_call(
        _tc_final,
        grid=grid,
        in_specs=[bs((BN, 48)), bs((BN, 48)), bfull((1, 40))],
        out_specs=bs((BN, 40)),
        out_shape=jax.ShapeDtypeStruct((N, 40), f32),
    )(acc2[0], acc2[1], b2.reshape(1, 40))
    return out


# speedup vs baseline: 2.6723x; 1.0924x over previous
import functools

import jax
import jax.numpy as jnp
from jax import lax
from jax.experimental import pallas as pl
from jax.experimental.pallas import tpu as pltpu
from jax.experimental.pallas import tpu_sc as plsc

N = 10000
E = 320000
F_IN = 128
HEADS = 8
HIDS = 8
NCLS = 40

NC = 2
NS = 16
NW = NC * NS
CK = 80
EW = E // NW
NCHUNK = EW // CK
ROWS_PER_TILE = N // NS

BN = 1000

NB = 4

_mesh = plsc.VectorSubcoreMesh(core_axis_name="c", subcore_axis_name="s")


def _pipeline(fire_gather, wait_gather, fire_scatter, wait_scatter, compute):
    assert NCHUNK % NB == 1
    for b in range(NB - 1):
        fire_gather(b, b)
    for j0 in range(NB):
        fire_gather(j0 + NB - 1, (j0 + NB - 1) % NB)
        wait_gather(j0, j0)
        compute(j0, j0)
        fire_scatter(j0, j0)

    def qbody(q, carry):
        j0 = NB * q
        for b in range(NB):
            j = j0 + b
            jn = jnp.minimum(j + NB - 1, NCHUNK - 1)
            fire_gather(jn, (b + NB - 1) % NB)
            wait_gather(j, b)
            wait_scatter(j, b)
            compute(j, b)
            fire_scatter(j, b)
        return carry

    lax.fori_loop(1, (NCHUNK - 1) // NB, qbody, 0)
    jt = NCHUNK - 1
    bt = jt % NB
    wait_gather(jt, bt)
    wait_scatter(jt, bt)
    compute(jt, bt)
    fire_scatter(jt, bt)
    for bb in range(1, NB - 1):
        wait_gather(jt, bb)
    for bb in range(NB):
        wait_scatter(jt, bb)
_sc_params = pltpu.CompilerParams(use_tc_tiling_on_sc=False,
                                  needs_layout_passes=False)


def _tc_prep1(x_ref, w1_ref, asrc_ref, adst_ref, st_ref, dt_ref):
    h = jnp.dot(x_ref[...], w1_ref[...], preferred_element_type=jnp.float32)
    a_s = jnp.dot(h, asrc_ref[...], preferred_element_type=jnp.float32)
    a_d = jnp.dot(h, adst_ref[...], preferred_element_type=jnp.float32)
    z = jnp.zeros((BN, 8), jnp.float32)
    st_ref[...] = jnp.concatenate([a_s, z, h], axis=1)
    dt_ref[...] = jnp.concatenate([a_d, z], axis=1)


def _tc_mid(a0_ref, a1_ref, b1_ref, w2_ref, as2_ref, ad2_ref, r8_ref,
            st_ref, d2_ref):
    a0 = a0_ref[...]
    a1 = a1_ref[...]
    den = a0[:, 0:8] + a1[:, 0:8]
    msum = a0[:, 16:80] + a1[:, 16:80]
    recip = 1.0 / (den + 1e-16)
    expand = jnp.dot(recip, r8_ref[...], preferred_element_type=jnp.float32)
    x1 = msum * expand + b1_ref[...]
    x1 = jnp.where(x1 > 0, x1, jnp.exp(x1) - 1.0)
    h2 = jnp.dot(x1, w2_ref[...], preferred_element_type=jnp.float32)
    s2 = jnp.sum(h2 * as2_ref[...], axis=1, keepdims=True)
    d2_ref[...] = jnp.sum(h2 * ad2_ref[...], axis=1, keepdims=True)
    st_ref[...] = jnp.concatenate(
        [jnp.ones((BN, 1), jnp.float32), s2, jnp.zeros((BN, 6), jnp.float32),
         h2], axis=1)


def _tc_final(a0_ref, a1_ref, b2_ref, out_ref):
    a0 = a0_ref[...]
    a1 = a1_ref[...]
    den = a0[:, 0:1] + a1[:, 0:1]
    msg = a0[:, 8:48] + a1[:, 8:48]
    out_ref[...] = msg / (den + 1e-16) + b2_ref[...]


def _zero_acc(acc, zbuf, r0, zcols):
    z16 = jnp.zeros((16,), jnp.float32)

    @plsc.parallel_loop(0, CK, unroll=4)
    def zb(e):
        for cb in range(zcols // 16):
            zbuf[e, pl.ds(cb * 16, 16)] = z16

    nfull = ROWS_PER_TILE // CK
    rem = ROWS_PER_TILE - nfull * CK
    for i in range(nfull):
        pltpu.sync_copy(zbuf, acc.at[pl.ds(r0 + i * CK, CK)])
    pltpu.sync_copy(zbuf.at[pl.ds(0, rem)],
                    acc.at[pl.ds(r0 + nfull * CK, rem)])


def _sc_layer1(st_ref, adt_ref, adj_ref, out_ref,
               acc, sbufs, cbufs, dbufs, idxs_v, idxd_v, gss, gds, sss):
    cid = lax.axis_index("c")
    sid = lax.axis_index("s")
    w = sid * NC + cid

    r0 = sid * ROWS_PER_TILE
    _zero_acc(acc, sbufs[0], r0, 80)
    pltpu.sync_copy(adj_ref.at[0, pl.ds(w * EW, EW)], idxs_v)
    pltpu.sync_copy(adj_ref.at[1, pl.ds(w * EW, EW)], idxd_v)
    plsc.subcore_barrier()

    iota = lax.iota(jnp.int32, 16)

    def fire_gather(j, b):
        pltpu.async_copy(st_ref.at[idxs_v.at[pl.ds(j * CK, CK)]],
                         sbufs[b], gss[b])
        pltpu.async_copy(adt_ref.at[idxd_v.at[pl.ds(j * CK, CK)]],
                         dbufs[b], gds[b])

    def wait_gather(j, b):
        pltpu.make_async_copy(st_ref.at[idxs_v.at[pl.ds(j * CK, CK)]],
                              sbufs[b], gss[b]).wait()
        pltpu.make_async_copy(adt_ref.at[idxd_v.at[pl.ds(j * CK, CK)]],
                              dbufs[b], gds[b]).wait()

    def fire_scatter(j, b):
        pltpu.async_copy(cbufs[b], acc.at[idxd_v.at[pl.ds(j * CK, CK)]],
                         sss[b], add=True)

    def wait_scatter(j, b):
        pltpu.make_async_copy(cbufs[b], acc.at[idxd_v.at[pl.ds(j * CK, CK)]],
                              sss[b]).wait()

    idx01 = iota >> 3

    def compute(j, b):
        sbuf, dbuf, cbuf = sbufs[b], dbufs[b], cbufs[b]

        @plsc.parallel_loop(0, CK, unroll=4)
        def edge_body(e):
            s0 = sbuf[e, pl.ds(0, 16)]
            d0 = dbuf[e, pl.ds(0, 16)]
            zz = s0 + d0
            ee = jnp.exp(jnp.maximum(zz, 0.2 * zz))
            cbuf[e, pl.ds(0, 16)] = ee
            for k in range(4):
                mk = ee.at[idx01 + 2 * k].get(mode="promise_in_bounds")
                hk = sbuf[e, pl.ds(16 + 16 * k, 16)]
                cbuf[e, pl.ds(16 + 16 * k, 16)] = hk * mk

    _pipeline(fire_gather, wait_gather, fire_scatter, wait_scatter, compute)
    plsc.subcore_barrier()
    pltpu.sync_copy(acc.at[pl.ds(r0, ROWS_PER_TILE)],
                    out_ref.at[cid, pl.ds(r0, ROWS_PER_TILE)])


def _sc_layer2(st_ref, ad2_ref, adj_ref, out_ref, acc, sbufs, cbufs,
               adst_v, idxs_v, idxd_v, gss, sss):
    cid = lax.axis_index("c")
    sid = lax.axis_index("s")
    w = sid * NC + cid

    r0 = sid * ROWS_PER_TILE
    _zero_acc(acc, sbufs[0], r0, 48)
    pltpu.sync_copy(ad2_ref, adst_v)
    pltpu.sync_copy(adj_ref.at[0, pl.ds(w * EW, EW)], idxs_v)
    pltpu.sync_copy(adj_ref.at[1, pl.ds(w * EW, EW)], idxd_v)
    plsc.subcore_barrier()

    def fire_gather(j, b):
        pltpu.async_copy(st_ref.at[idxs_v.at[pl.ds(j * CK, CK)]],
                         sbufs[b], gss[b])

    def wait_gather(j, b):
        pltpu.make_async_copy(st_ref.at[idxs_v.at[pl.ds(j * CK, CK)]],
                              sbufs[b], gss[b]).wait()

    def fire_scatter(j, b):
        pltpu.async_copy(cbufs[b], acc.at[idxd_v.at[pl.ds(j * CK, CK)]],
                         sss[b], add=True)

    def wait_scatter(j, b):
        pltpu.make_async_copy(cbufs[b], acc.at[idxd_v.at[pl.ds(j * CK, CK)]],
                              sss[b]).wait()

    iota = lax.iota(jnp.int32, 16)
    one16 = jnp.ones((16,), jnp.int32)

    def compute(j, b):
        sbuf, cbuf = sbufs[b], cbufs[b]
        for g in range(5):
            local = iota + g * 16
            id_g = idxd_v[pl.ds(j * CK + g * 16, 16)]
            a_s = plsc.load_gather(sbuf, [local, one16])
            a_d = plsc.load_gather(adst_v, [id_g])
            zz = a_s + a_d
            ee16 = jnp.exp(jnp.maximum(zz, 0.2 * zz))

            @plsc.parallel_loop(0, 16, unroll=4)
            def edge_body(i):
                e = g * 16 + i
                ii = jnp.full((16,), i, jnp.int32)
                mult = ee16.at[ii].get(mode="promise_in_bounds")
                for k in range(3):
                    rk = sbuf[e, pl.ds(16 * k, 16)]
                    cbuf[e, pl.ds(16 * k, 16)] = rk * mult

    _pipeline(fire_gather, wait_gather, fire_scatter, wait_scatter, compute)
    plsc.subcore_barrier()
    pltpu.sync_copy(acc.at[pl.ds(r0, ROWS_PER_TILE)],
                    out_ref.at[cid, pl.ds(r0, ROWS_PER_TILE)])


def kernel(data, adj, W1, a_src1, a_dst1, b1, W2, a_src2, a_dst2, b2):
    f32 = jnp.float32
    eye8 = jnp.eye(8, dtype=f32)
    A_src = (eye8[:, None, :] * a_src1[:, :, None]).reshape(64, 8)
    A_dst = (eye8[:, None, :] * a_dst1[:, :, None]).reshape(64, 8)
    R8 = jnp.repeat(eye8, 8, axis=1)

    grid = (N // BN,)
    bs = lambda shp: pl.BlockSpec(shp, lambda i: (i, 0))
    bfull = lambda shp: pl.BlockSpec(shp, lambda i: (0, 0))

    st1, adst1 = pl.pallas_call(
        _tc_prep1,
        grid=grid,
        in_specs=[bs((BN, F_IN)), bfull((F_IN, 64)), bfull((64, 8)),
                  bfull((64, 8))],
        out_specs=[bs((BN, 80)), bs((BN, 16))],
        out_shape=[jax.ShapeDtypeStruct((N, 80), f32),
                   jax.ShapeDtypeStruct((N, 16), f32)],
    )(data, W1, A_src, A_dst)

    sc1 = pl.kernel(
        _sc_layer1,
        out_type=jax.ShapeDtypeStruct((2, N, 80), f32),
        mesh=_mesh,
        compiler_params=_sc_params,
        scratch_types=[
            pltpu.VMEM_SHARED((N, 80), f32),
            [pltpu.VMEM((CK, 80), f32)] * NB,
            [pltpu.VMEM((CK, 80), f32)] * NB,
            [pltpu.VMEM((CK, 16), f32)] * NB,
            pltpu.VMEM((EW,), jnp.int32),
            pltpu.VMEM((EW,), jnp.int32),
            [pltpu.SemaphoreType.DMA] * NB,
            [pltpu.SemaphoreType.DMA] * NB,
            [pltpu.SemaphoreType.DMA] * NB,
        ],
    )
    acc1 = sc1(st1, adst1, adj)

    st2, adst2 = pl.pallas_call(
        _tc_mid,
        grid=grid,
        in_specs=[bs((BN, 80)), bs((BN, 80)), bfull((1, 64)), bfull((64, 40)),
                  bfull((1, 40)), bfull((1, 40)), bfull((8, 64))],
        out_specs=[bs((BN, 48)), bs((BN, 1))],
        out_shape=[jax.ShapeDtypeStruct((N, 48), f32),
                   jax.ShapeDtypeStruct((N, 1), f32)],
    )(acc1[0], acc1[1], b1.reshape(1, 64), W2, a_src2, a_dst2, R8)

    sc2 = pl.kernel(
        _sc_layer2,
        out_type=jax.ShapeDtypeStruct((2, N, 48), f32),
        mesh=_mesh,
        compiler_params=_sc_params,
        scratch_types=[
            pltpu.VMEM_SHARED((N, 48), f32),
            [pltpu.VMEM((CK, 48), f32)] * NB,
            [pltpu.VMEM((CK, 48), f32)] * NB,
            pltpu.VMEM((N,), f32),
            pltpu.VMEM((EW,), jnp.int32),
            pltpu.VMEM((EW,), jnp.int32),
            [pltpu.SemaphoreType.DMA] * NB,
            [pltpu.SemaphoreType.DMA] * NB,
        ],
    )
    acc2 = sc2(st2, adst2.reshape(N), adj)

    out = pl.pallas_call(
        _tc_final,
        grid=grid,
        in_specs=[bs((BN, 48)), bs((BN, 48)), bfull((1, 40))],
        out_specs=bs((BN, 40)),
        out_shape=jax.ShapeDtypeStruct((N, 40), f32),
    )(acc2[0], acc2[1], b2.reshape(1, 40))
    return out
